# baseline (speedup 1.0000x reference)
import functools

import jax
import jax.numpy as jnp
from jax import lax
from jax.experimental import pallas as pl
from jax.experimental.pallas import tpu as pltpu
from jax.experimental.pallas import tpu_sc as plsc

NUM_IMAGES = 100000
EMBED_DIM = 32
BATCH = 16384

_info = plsc.get_sparse_core_info()
_NC, _NS = _info.num_cores, _info.num_subcores
_NW = _NC * _NS
_LANE_TILES = (NUM_IMAGES + 127) // 128
_FULL_TILES = NUM_IMAGES // 128
_BASE_W = _LANE_TILES // _NW
_EXTRA = _LANE_TILES - _BASE_W * _NW
_MAX_W = _BASE_W + 1
_NVEC = BATCH // 16
_NSLOT = 16


@functools.partial(
    pl.kernel,
    mesh=plsc.VectorSubcoreMesh(core_axis_name="c", subcore_axis_name="s"),
    out_type=jax.ShapeDtypeStruct((BATCH * EMBED_DIM,), jnp.float32),
    scratch_types=[
        pltpu.VMEM((4, _MAX_W, 8, 128), jnp.float32),
        pltpu.VMEM((BATCH,), jnp.int32),
        pltpu.VMEM((32,), jnp.int32),
        pltpu.VMEM((_NSLOT, EMBED_DIM), jnp.float32),
        pltpu.SemaphoreType.DMA,
        pltpu.SemaphoreType.DMA((_NSLOT,)),
    ],
    compiler_params=pltpu.CompilerParams(use_tc_tiling_on_sc=True, needs_layout_passes=False, disable_bounds_checks=True),
)
def _lookup_kernel(table_t, idx_hbm, out_hbm, block_v, idx_v, cstage_v,
                   rows_v, sem_in, sem_out):
    wid = lax.axis_index("s") * _NC + lax.axis_index("c")
    c0 = wid * _BASE_W + jnp.minimum(wid, _EXTRA)
    wc = jnp.where(wid < _EXTRA, _BASE_W + 1, _BASE_W)
    lo = c0 * 128
    hi = (c0 + wc) * 128

    iota = lax.iota(jnp.int32, 16)
    d_hi = iota // 8
    s_hi = iota % 8

    def issue(ct, do_issue):
        c_tile = c0 + ct
        for d in range(4):
            cp = pltpu.make_async_copy(
                table_t.at[pl.ds(d * 8, 8), pl.ds(c_tile * 128, 128)],
                block_v.at[d, ct],
                sem_in,
            )
            if do_issue:
                cp.start()
            else:
                cp.wait()
        return ct + 1

    lax.fori_loop(0, wc, lambda ct, _: issue(ct, True), 0)
    idx_cp = pltpu.make_async_copy(idx_hbm, idx_v, sem_in)
    idx_cp.start()
    lax.fori_loop(0, wc, lambda ct, _: issue(ct, False), 0)
    idx_cp.wait()

    def hit_body(carry):
        m, v, i = carry
        pos_v = plsc.all_reduce_ffs(m)
        pos = pos_v[0]
        c = cstage_v[pl.ds(pos, 16)][0]
        j = v * 16 + pos
        qt = (c >> 7) - c0
        ql = c & 127
        slot = i % _NSLOT
        slot_ref = rows_v.at[slot]

        @pl.when(i >= _NSLOT)
        def _():
            pltpu.make_async_copy(
                out_hbm.at[pl.ds(0, EMBED_DIM)], slot_ref, sem_out.at[slot]
            ).wait()

        qt_v = jnp.full((16,), qt, jnp.int32)
        ql_v = jnp.full((16,), ql, jnp.int32)
        lo16 = plsc.load_gather(block_v, [d_hi, qt_v, s_hi, ql_v])
        hi16 = plsc.load_gather(block_v, [d_hi + 2, qt_v, s_hi, ql_v])
        slot_ref[pl.ds(0, 16)] = lo16
        slot_ref[pl.ds(16, 16)] = hi16
        pltpu.make_async_copy(
            slot_ref, out_hbm.at[pl.ds(j * EMBED_DIM, EMBED_DIM)],
            sem_out.at[slot],
        ).start()
        return m & (iota != pos_v), v, i + 1

    def scan_body(v, i):
        c_vec = idx_v[pl.ds(v * 16, 16)]
        lo_v = jnp.full((16,), lo, jnp.int32)
        hi_v = jnp.full((16,), hi, jnp.int32)
        m = (c_vec >= lo_v) & (c_vec < hi_v)
        cstage_v[pl.ds(0, 16)] = c_vec
        _, _, i = lax.while_loop(
            lambda carry: jnp.any(carry[0]), hit_body, (m, v, i)
        )
        return i

    total = lax.fori_loop(0, _NVEC, scan_body, jnp.int32(0))

    for s in range(_NSLOT):
        @pl.when(total > s)
        def _():
            pltpu.make_async_copy(
                out_hbm.at[pl.ds(0, EMBED_DIM)], rows_v.at[s], sem_out.at[s]
            ).wait()


def kernel(camera_indices, embedding_weight):
    idx = camera_indices.astype(jnp.int32)
    flat = _lookup_kernel(embedding_weight.T, idx)
    return flat.reshape(BATCH, EMBED_DIM)
